# Initial kernel scaffold; baseline (speedup 1.0000x reference)
import functools

import jax
import jax.numpy as jnp
from jax import lax
from jax.experimental import pallas as pl
from jax.experimental.pallas import tpu as pltpu
from jax.experimental.pallas import tpu_sc as plsc

N_CORES = 2
N_SUBCORES = 16
NW = N_CORES * N_SUBCORES
LANES = 16
BLK = 128
H = 4
C = 32
HC = H * C
NEG_SLOPE = 0.2
EPS = 1e-16


def _project_body(x_ref, wl_ref, bl_ref, wr_ref, br_ref, xl_ref, xr_ref):
    xb = x_ref[...]
    xl_ref[...] = (
        jnp.dot(xb, wl_ref[...], preferred_element_type=jnp.float32) + bl_ref[...]
    )
    xr_ref[...] = (
        jnp.dot(xb, wr_ref[...], preferred_element_type=jnp.float32) + br_ref[...]
    )


def _project(x, W_l, b_l, W_r, b_r, bn):
    n, d = x.shape
    return pl.pallas_call(
        _project_body,
        grid=(n // bn,),
        in_specs=[
            pl.BlockSpec((bn, d), lambda i: (i, 0)),
            pl.BlockSpec((d, HC), lambda i: (0, 0)),
            pl.BlockSpec((HC,), lambda i: (0,)),
            pl.BlockSpec((d, HC), lambda i: (0, 0)),
            pl.BlockSpec((HC,), lambda i: (0,)),
        ],
        out_specs=[
            pl.BlockSpec((bn, HC), lambda i: (i, 0)),
            pl.BlockSpec((bn, HC), lambda i: (i, 0)),
        ],
        out_shape=[jax.ShapeDtypeStruct((n, HC), jnp.float32)] * 2,
    )(x, W_l, b_l, W_r, b_r)


def _edge_pass(xl, xr, src3, dst3, att_flat, z_den, z_out, n, nblk, etot):
    chunk = nblk * BLK
    rows_out = n // N_SUBCORES
    den_tiles = n // 1000

    mesh = plsc.VectorSubcoreMesh(core_axis_name="c", subcore_axis_name="s")

    @functools.partial(
        pl.kernel,
        out_type=[
            jax.ShapeDtypeStruct((N_CORES, n, HC), jnp.float32),
            jax.ShapeDtypeStruct((N_CORES, n, H), jnp.float32),
        ],
        mesh=mesh,
        scratch_types=[
            pltpu.VMEM((nblk, BLK), jnp.int32),
            pltpu.VMEM((nblk, BLK), jnp.int32),
            pltpu.VMEM((BLK, HC), jnp.float32),
            pltpu.VMEM((BLK, HC), jnp.float32),
            pltpu.VMEM((BLK, HC), jnp.float32),
            pltpu.VMEM((BLK, H), jnp.float32),
            pltpu.VMEM((HC,), jnp.float32),
            pltpu.VMEM_SHARED((n, HC), jnp.float32),
            pltpu.VMEM_SHARED((n, H), jnp.float32),
            pltpu.SemaphoreType.DMA,
            pltpu.SemaphoreType.DMA,
        ],
    )
    def body(xl_hbm, xr_hbm, src_hbm, dst_hbm, att_hbm, zden_hbm, zout_hbm,
             part_hbm, den_hbm,
             src_c, dst_c, xl_s, xr_s, out_s, ea_s, att_v, out_sh, den_sh,
             sem1, sem2):
        cid = lax.axis_index("c")
        sid = lax.axis_index("s")
        wid = sid * N_CORES + cid

        pltpu.sync_copy(zout_hbm, out_sh.at[pl.ds(sid * rows_out, rows_out)])

        @pl.when(sid < den_tiles)
        def _():
            pltpu.sync_copy(zden_hbm, den_sh.at[pl.ds(sid * 1000, 1000)])

        pltpu.sync_copy(src_hbm.at[wid], src_c)
        pltpu.sync_copy(dst_hbm.at[wid], dst_c)
        pltpu.sync_copy(att_hbm, att_v)
        plsc.subcore_barrier()

        base = wid * chunk
        iota16 = lax.iota(jnp.int32, LANES)

        def blk_body(b, carry):
            src_v = src_c.at[b]
            dst_v = dst_c.at[b]
            cp1 = pltpu.async_copy(xl_hbm.at[src_v], xl_s, sem1)
            cp2 = pltpu.async_copy(xr_hbm.at[dst_v], xr_s, sem2)
            cp1.wait()
            cp2.wait()

            def sub_body(t, carry2):
                rows = t * LANES + iota16
                gids = base + b * BLK + rows
                valid = gids < etot
                for h in range(H):
                    acc = jnp.zeros((LANES,), jnp.float32)
                    for cc in range(C):
                        f = h * C + cc
                        colv = jnp.full((LANES,), f, jnp.int32)
                        z = (plsc.load_gather(xl_s, [rows, colv])
                             + plsc.load_gather(xr_s, [rows, colv]))
                        z = jnp.maximum(z, NEG_SLOPE * z)
                        acc = acc + z * att_v[f]
                    ea = jnp.exp(acc)
                    ea = jnp.where(valid, ea, 0.0)
                    plsc.store_scatter(
                        ea_s, [rows, jnp.full((LANES,), h, jnp.int32)], ea)
                    for cc in range(C):
                        f = h * C + cc
                        colv = jnp.full((LANES,), f, jnp.int32)
                        v = plsc.load_gather(xl_s, [rows, colv]) * ea
                        plsc.store_scatter(out_s, [rows, colv], v)
                return carry2

            lax.fori_loop(0, BLK // LANES, sub_body, 0)
            pltpu.sync_copy(out_s, out_sh.at[dst_v], add=True)
            pltpu.sync_copy(ea_s, den_sh.at[dst_v], add=True)
            return carry

        lax.fori_loop(0, nblk, blk_body, 0)
        plsc.subcore_barrier()

        pltpu.sync_copy(
            out_sh.at[pl.ds(sid * rows_out, rows_out)],
            part_hbm.at[cid, pl.ds(sid * rows_out, rows_out), :])

        @pl.when(sid < den_tiles)
        def _():
            pltpu.sync_copy(
                den_sh.at[pl.ds(sid * 1000, 1000)],
                den_hbm.at[cid, pl.ds(sid * 1000, 1000), :])

    return body(xl, xr, src3, dst3, att_flat, z_den, z_out)


def _finalize_body(p0_ref, p1_ref, d0_ref, d1_ref, esel_ref, bias_ref, o_ref):
    den = d0_ref[...] + d1_ref[...]
    den_exp = jnp.dot(den, esel_ref[...], preferred_element_type=jnp.float32)
    o_ref[...] = (p0_ref[...] + p1_ref[...]) / (den_exp + EPS) + bias_ref[...]


def _finalize(p0, p1, d0, d1, esel, bias, bn):
    n = p0.shape[0]
    return pl.pallas_call(
        _finalize_body,
        grid=(n // bn,),
        in_specs=[
            pl.BlockSpec((bn, HC), lambda i: (i, 0)),
            pl.BlockSpec((bn, HC), lambda i: (i, 0)),
            pl.BlockSpec((bn, H), lambda i: (i, 0)),
            pl.BlockSpec((bn, H), lambda i: (i, 0)),
            pl.BlockSpec((H, HC), lambda i: (0, 0)),
            pl.BlockSpec((HC,), lambda i: (0,)),
        ],
        out_specs=pl.BlockSpec((bn, HC), lambda i: (i, 0)),
        out_shape=jax.ShapeDtypeStruct((n, HC), jnp.float32),
    )(p0, p1, d0, d1, esel, bias)


def kernel(x, edge_index, W_l, b_l, W_r, b_r, att, bias):
    n, d = x.shape
    e = edge_index.shape[1]
    etot = e + n
    nblk = -(-etot // (NW * BLK))
    chunk = nblk * BLK
    epad = NW * chunk

    idt = edge_index.dtype
    loop_idx = jnp.arange(n, dtype=idt)
    pad = jnp.zeros((epad - etot,), idt)
    src3 = jnp.concatenate([edge_index[0], loop_idx, pad]).reshape(NW, nblk, BLK)
    dst3 = jnp.concatenate([edge_index[1], loop_idx, pad]).reshape(NW, nblk, BLK)

    xl, xr = _project(x, W_l, b_l, W_r, b_r, bn=1000)

    z_den = jnp.zeros((1000, H), jnp.float32)
    z_out = jnp.zeros((n // N_SUBCORES, HC), jnp.float32)
    part, den = _edge_pass(xl, xr, src3, dst3, att.reshape(HC), z_den, z_out,
                           n, nblk, etot)

    esel = (jnp.arange(HC, dtype=jnp.int32)[None, :] // C
            == jnp.arange(H, dtype=jnp.int32)[:, None]).astype(jnp.float32)
    return _finalize(part[0], part[1], den[0], den[1], esel, bias, bn=1000)


# speedup vs baseline: 12.1763x; 12.1763x over previous
import functools

import jax
import jax.numpy as jnp
from jax import lax
from jax.experimental import pallas as pl
from jax.experimental.pallas import tpu as pltpu
from jax.experimental.pallas import tpu_sc as plsc

N_CORES = 2
N_SUBCORES = 16
NW = N_CORES * N_SUBCORES
LANES = 16
BLK = 64
H = 4
C = 32
HC = H * C
NEG_SLOPE = 0.2
EPS = 1e-16
DEN_W = 16


def _project_body(x_ref, wl_ref, bl_ref, wr_ref, br_ref, xl_ref, xr_ref):
    xb = x_ref[...]
    xl_ref[...] = (
        jnp.dot(xb, wl_ref[...], preferred_element_type=jnp.float32) + bl_ref[...]
    )
    xr_ref[...] = (
        jnp.dot(xb, wr_ref[...], preferred_element_type=jnp.float32) + br_ref[...]
    )


def _project(x, W_l, b_l, W_r, b_r, bn):
    n, d = x.shape
    return pl.pallas_call(
        _project_body,
        grid=(n // bn,),
        in_specs=[
            pl.BlockSpec((bn, d), lambda i: (i, 0)),
            pl.BlockSpec((d, HC), lambda i: (0, 0)),
            pl.BlockSpec((HC,), lambda i: (0,)),
            pl.BlockSpec((d, HC), lambda i: (0, 0)),
            pl.BlockSpec((HC,), lambda i: (0,)),
        ],
        out_specs=[
            pl.BlockSpec((bn, HC), lambda i: (i, 0)),
            pl.BlockSpec((bn, HC), lambda i: (i, 0)),
        ],
        out_shape=[jax.ShapeDtypeStruct((n, HC), jnp.float32)] * 2,
    )(x, W_l, b_l, W_r, b_r)


def _edge_pass(xl, xr, src3, dst3, att_flat, z_den, z_out, n, nblk, etot):
    chunk = nblk * BLK
    den_tiles = n // 1000

    mesh = plsc.VectorSubcoreMesh(core_axis_name="c", subcore_axis_name="s")

    @functools.partial(
        pl.kernel,
        out_type=[
            jax.ShapeDtypeStruct((N_CORES, n, HC), jnp.float32),
            jax.ShapeDtypeStruct((N_CORES, n, DEN_W), jnp.float32),
        ],
        mesh=mesh,
        compiler_params=pltpu.CompilerParams(
            needs_layout_passes=False, use_tc_tiling_on_sc=False),
        scratch_types=[
            pltpu.VMEM((BLK,), jnp.int32),
            pltpu.VMEM((BLK,), jnp.int32),
            pltpu.VMEM((BLK, HC), jnp.float32),
            pltpu.VMEM((BLK, HC), jnp.float32),
            pltpu.VMEM((BLK, DEN_W), jnp.float32),
            pltpu.VMEM((HC,), jnp.float32),
            pltpu.VMEM_SHARED((n, HC), jnp.float32),
            pltpu.VMEM_SHARED((n, DEN_W), jnp.float32),
            pltpu.SemaphoreType.DMA,
            pltpu.SemaphoreType.DMA,
        ],
    )
    def body(xl_hbm, xr_hbm, src_hbm, dst_hbm, att_hbm, zden_hbm, zout_hbm,
             part_hbm, den_hbm,
             src_v, dst_v, xl_s, xr_s, ea_s, att_v, out_sh, den_sh,
             sem1, sem2):
        cid = lax.axis_index("c")
        sid = lax.axis_index("s")
        wid = sid * N_CORES + cid

        pltpu.sync_copy(zout_hbm, out_sh.at[pl.ds(sid * 624, 640)])
        pltpu.sync_copy(zden_hbm, den_sh.at[pl.ds(sid * 624, 640)])

        pltpu.sync_copy(att_hbm, att_v)

        def zero_ea(i, carry):
            ea_s[i, :] = jnp.zeros((DEN_W,), jnp.float32)
            return carry
        lax.fori_loop(0, BLK, zero_ea, 0)
        plsc.subcore_barrier()

        base = wid * chunk
        iota16 = lax.iota(jnp.int32, LANES)

        def blk_body(b, carry):
            pltpu.sync_copy(src_hbm.at[wid, b], src_v)
            pltpu.sync_copy(dst_hbm.at[wid, b], dst_v)
            cp1 = pltpu.async_copy(xl_hbm.at[src_v], xl_s, sem1)
            cp2 = pltpu.async_copy(xr_hbm.at[dst_v], xr_s, sem2)
            cp1.wait()
            cp2.wait()

            def sub_body(t, carry2):
                rows = t * LANES + iota16
                gids = base + b * BLK + rows
                valid = gids < etot
                for h in range(H):
                    av0 = att_v[pl.ds(h * C, LANES)]
                    av1 = att_v[pl.ds(h * C + LANES, LANES)]
                    acc = jnp.zeros((LANES,), jnp.float32)
                    for cc in range(C):
                        f = h * C + cc
                        att_s = av0[cc] if cc < LANES else av1[cc - LANES]
                        colv = jnp.full((LANES,), f, jnp.int32)
                        z = (plsc.load_gather(xl_s, [rows, colv])
                             + plsc.load_gather(xr_s, [rows, colv]))
                        z = jnp.maximum(z, NEG_SLOPE * z)
                        acc = acc + z * att_s
                    ea = jnp.exp(acc)
                    ea = jnp.where(valid, ea, 0.0)
                    plsc.store_scatter(
                        ea_s, [rows, jnp.full((LANES,), h, jnp.int32)], ea)
                    for cc in range(C):
                        f = h * C + cc
                        colv = jnp.full((LANES,), f, jnp.int32)
                        v = plsc.load_gather(xl_s, [rows, colv]) * ea
                        plsc.store_scatter(xl_s, [rows, colv], v)
                return carry2

            lax.fori_loop(0, BLK // LANES, sub_body, 0)
            pltpu.sync_copy(xl_s, out_sh.at[dst_v], add=True)
            pltpu.sync_copy(ea_s, den_sh.at[dst_v], add=True)
            return carry

        lax.fori_loop(0, nblk, blk_body, 0)
        plsc.subcore_barrier()

        pltpu.sync_copy(
            out_sh.at[pl.ds(sid * 624, 640)],
            part_hbm.at[cid, pl.ds(sid * 624, 640), :])
        pltpu.sync_copy(
            den_sh.at[pl.ds(sid * 624, 640)],
            den_hbm.at[cid, pl.ds(sid * 624, 640), :])

    return body(xl, xr, src3, dst3, att_flat, z_den, z_out)


def _finalize_body(p0_ref, p1_ref, d0_ref, d1_ref, esel_ref, bias_ref, o_ref):
    den = d0_ref[...] + d1_ref[...]
    den_exp = jnp.dot(den, esel_ref[...], preferred_element_type=jnp.float32)
    o_ref[...] = (p0_ref[...] + p1_ref[...]) / (den_exp + EPS) + bias_ref[...]


def _finalize(p0, p1, d0, d1, esel, bias, bn):
    n = p0.shape[0]
    return pl.pallas_call(
        _finalize_body,
        grid=(n // bn,),
        in_specs=[
            pl.BlockSpec((bn, HC), lambda i: (i, 0)),
            pl.BlockSpec((bn, HC), lambda i: (i, 0)),
            pl.BlockSpec((bn, DEN_W), lambda i: (i, 0)),
            pl.BlockSpec((bn, DEN_W), lambda i: (i, 0)),
            pl.BlockSpec((DEN_W, HC), lambda i: (0, 0)),
            pl.BlockSpec((HC,), lambda i: (0,)),
        ],
        out_specs=pl.BlockSpec((bn, HC), lambda i: (i, 0)),
        out_shape=jax.ShapeDtypeStruct((n, HC), jnp.float32),
    )(p0, p1, d0, d1, esel, bias)


def kernel(x, edge_index, W_l, b_l, W_r, b_r, att, bias):
    n, d = x.shape
    e = edge_index.shape[1]
    etot = e + n
    nblk = -(-etot // (NW * BLK))
    chunk = nblk * BLK
    epad = NW * chunk

    idt = edge_index.dtype
    loop_idx = jnp.arange(n, dtype=idt)
    pad = jnp.zeros((epad - etot,), idt)
    src3 = jnp.concatenate([edge_index[0], loop_idx, pad]).reshape(NW, nblk, BLK)
    dst3 = jnp.concatenate([edge_index[1], loop_idx, pad]).reshape(NW, nblk, BLK)

    xl, xr = _project(x, W_l, b_l, W_r, b_r, bn=1000)

    z_den = jnp.zeros((640, DEN_W), jnp.float32)
    z_out = jnp.zeros((640, HC), jnp.float32)
    part, den = _edge_pass(xl, xr, src3, dst3, att.reshape(HC), z_den, z_out,
                           n, nblk, etot)

    esel = (jnp.arange(HC, dtype=jnp.int32)[None, :] // C
            == jnp.arange(DEN_W, dtype=jnp.int32)[:, None]).astype(jnp.float32)
    return _finalize(part[0], part[1], den[0], den[1], esel, bias, bn=1000)


# speedup vs baseline: 14.9077x; 1.2243x over previous
import functools

import jax
import jax.numpy as jnp
from jax import lax
from jax.experimental import pallas as pl
from jax.experimental.pallas import tpu as pltpu
from jax.experimental.pallas import tpu_sc as plsc

N_CORES = 2
N_SUBCORES = 16
NW = N_CORES * N_SUBCORES
LANES = 16
BLK = 64
H = 4
C = 32
HC = H * C
NEG_SLOPE = 0.2
EPS = 1e-16
DEN_W = 16


def _project_body(x_ref, wl_ref, bl_ref, wr_ref, br_ref, xl_ref, xr_ref):
    xb = x_ref[...]
    xl_ref[...] = (
        jnp.dot(xb, wl_ref[...], preferred_element_type=jnp.float32) + bl_ref[...]
    )
    xr_ref[...] = (
        jnp.dot(xb, wr_ref[...], preferred_element_type=jnp.float32) + br_ref[...]
    )


def _project(x, W_l, b_l, W_r, b_r, bn):
    n, d = x.shape
    return pl.pallas_call(
        _project_body,
        grid=(n // bn,),
        in_specs=[
            pl.BlockSpec((bn, d), lambda i: (i, 0)),
            pl.BlockSpec((d, HC), lambda i: (0, 0)),
            pl.BlockSpec((HC,), lambda i: (0,)),
            pl.BlockSpec((d, HC), lambda i: (0, 0)),
            pl.BlockSpec((HC,), lambda i: (0,)),
        ],
        out_specs=[
            pl.BlockSpec((bn, HC), lambda i: (i, 0)),
            pl.BlockSpec((bn, HC), lambda i: (i, 0)),
        ],
        out_shape=[jax.ShapeDtypeStruct((n, HC), jnp.float32)] * 2,
    )(x, W_l, b_l, W_r, b_r)


def _edge_pass(xl, xr, src3, dst3, att_flat, z_den, z_out, n, nblk, etot):
    chunk = nblk * BLK
    den_tiles = n // 1000

    mesh = plsc.VectorSubcoreMesh(core_axis_name="c", subcore_axis_name="s")

    @functools.partial(
        pl.kernel,
        out_type=[
            jax.ShapeDtypeStruct((N_CORES, n, HC), jnp.float32),
            jax.ShapeDtypeStruct((N_CORES, n, DEN_W), jnp.float32),
        ],
        mesh=mesh,
        compiler_params=pltpu.CompilerParams(
            needs_layout_passes=False, use_tc_tiling_on_sc=False),
        scratch_types=[
            pltpu.VMEM((BLK,), jnp.int32),
            pltpu.VMEM((BLK,), jnp.int32),
            pltpu.VMEM((BLK, HC), jnp.float32),
            pltpu.VMEM((BLK, DEN_W), jnp.float32),
            pltpu.VMEM((HC,), jnp.float32),
            pltpu.VMEM_SHARED((n, HC), jnp.float32),
            pltpu.VMEM_SHARED((n, DEN_W), jnp.float32),
            pltpu.SemaphoreType.DMA,
            pltpu.SemaphoreType.DMA,
        ],
    )
    def body(xl_hbm, xr_hbm, src_hbm, dst_hbm, att_hbm, zden_hbm, zout_hbm,
             part_hbm, den_hbm,
             src_v, dst_v, xm_s, ea_s, att_v, out_sh, den_sh,
             sem1, sem2):
        cid = lax.axis_index("c")
        sid = lax.axis_index("s")
        wid = sid * N_CORES + cid

        pltpu.sync_copy(zout_hbm, out_sh.at[pl.ds(sid * 624, 640)])
        pltpu.sync_copy(zden_hbm, den_sh.at[pl.ds(sid * 624, 640)])

        pltpu.sync_copy(att_hbm, att_v)

        def zero_ea(i, carry):
            ea_s[i, :] = jnp.zeros((DEN_W,), jnp.float32)
            return carry
        lax.fori_loop(0, BLK, zero_ea, 0)
        plsc.subcore_barrier()

        base = wid * chunk
        iota16 = lax.iota(jnp.int32, LANES)

        def blk_body(b, carry):
            ci1 = pltpu.async_copy(src_hbm.at[wid, b], src_v, sem1)
            ci2 = pltpu.async_copy(dst_hbm.at[wid, b], dst_v, sem2)
            ci1.wait()
            ci2.wait()
            pltpu.sync_copy(xl_hbm.at[src_v], xm_s)
            pltpu.sync_copy(xr_hbm.at[dst_v], xm_s, add=True)

            def sub_body(t, carry2):
                rows = t * LANES + iota16
                gids = base + b * BLK + rows
                valid = gids < etot
                for h in range(H):
                    av0 = att_v[pl.ds(h * C, LANES)]
                    av1 = att_v[pl.ds(h * C + LANES, LANES)]
                    acc = jnp.zeros((LANES,), jnp.float32)
                    for cc in range(C):
                        f = h * C + cc
                        att_s = av0[cc] if cc < LANES else av1[cc - LANES]
                        colv = jnp.full((LANES,), f, jnp.int32)
                        z = plsc.load_gather(xm_s, [rows, colv])
                        z = jnp.maximum(z, NEG_SLOPE * z)
                        acc = acc + z * att_s
                    ea = jnp.exp(acc)
                    ea = jnp.where(valid, ea, 0.0)
                    plsc.store_scatter(
                        ea_s, [rows, jnp.full((LANES,), h, jnp.int32)], ea)
                    for cc in range(C):
                        f = h * C + cc
                        colv = jnp.full((LANES,), f, jnp.int32)
                        v = plsc.load_gather(xm_s, [rows, colv]) * ea
                        plsc.store_scatter(xm_s, [rows, colv], v)
                return carry2

            lax.fori_loop(0, BLK // LANES, sub_body, 0)
            cs1 = pltpu.async_copy(xm_s, out_sh.at[dst_v], sem1, add=True)
            cs2 = pltpu.async_copy(ea_s, den_sh.at[dst_v], sem2, add=True)
            cs1.wait()
            cs2.wait()
            return carry

        lax.fori_loop(0, nblk, blk_body, 0)
        plsc.subcore_barrier()

        pltpu.sync_copy(
            out_sh.at[pl.ds(sid * 624, 640)],
            part_hbm.at[cid, pl.ds(sid * 624, 640), :])
        pltpu.sync_copy(
            den_sh.at[pl.ds(sid * 624, 640)],
            den_hbm.at[cid, pl.ds(sid * 624, 640), :])

    return body(xl, xr, src3, dst3, att_flat, z_den, z_out)


def _finalize_body(p0_ref, p1_ref, d0_ref, d1_ref, xr_ref, bias_ref, o_ref):
    den = d0_ref[...] + d1_ref[...]
    bn = den.shape[0]
    den_exp = jnp.concatenate(
        [jnp.broadcast_to(den[:, h][:, None], (bn, C)) for h in range(H)],
        axis=1)
    num = p0_ref[...] + p1_ref[...] - xr_ref[...] * den_exp
    o_ref[...] = num / (den_exp + EPS) + bias_ref[...]


def _finalize(p0, p1, d0, d1, xr, bias, bn):
    n = p0.shape[0]
    return pl.pallas_call(
        _finalize_body,
        grid=(n // bn,),
        in_specs=[
            pl.BlockSpec((bn, HC), lambda i: (i, 0)),
            pl.BlockSpec((bn, HC), lambda i: (i, 0)),
            pl.BlockSpec((bn, DEN_W), lambda i: (i, 0)),
            pl.BlockSpec((bn, DEN_W), lambda i: (i, 0)),
            pl.BlockSpec((bn, HC), lambda i: (i, 0)),
            pl.BlockSpec((HC,), lambda i: (0,)),
        ],
        out_specs=pl.BlockSpec((bn, HC), lambda i: (i, 0)),
        out_shape=jax.ShapeDtypeStruct((n, HC), jnp.float32),
    )(p0, p1, d0, d1, xr, bias)


def kernel(x, edge_index, W_l, b_l, W_r, b_r, att, bias):
    n, d = x.shape
    e = edge_index.shape[1]
    etot = e + n
    nblk = -(-etot // (NW * BLK))
    chunk = nblk * BLK
    epad = NW * chunk

    idt = edge_index.dtype
    loop_idx = jnp.arange(n, dtype=idt)
    pad = jnp.zeros((epad - etot,), idt)
    src3 = jnp.concatenate([edge_index[0], loop_idx, pad]).reshape(NW, nblk, BLK)
    dst3 = jnp.concatenate([edge_index[1], loop_idx, pad]).reshape(NW, nblk, BLK)

    xl, xr = _project(x, W_l, b_l, W_r, b_r, bn=1000)

    z_den = jnp.zeros((640, DEN_W), jnp.float32)
    z_out = jnp.zeros((640, HC), jnp.float32)
    part, den = _edge_pass(xl, xr, src3, dst3, att.reshape(HC), z_den, z_out,
                           n, nblk, etot)

    return _finalize(part[0], part[1], den[0], den[1], xr, bias, bn=1000)


# speedup vs baseline: 15.5524x; 1.0432x over previous
import functools

import jax
import jax.numpy as jnp
from jax import lax
from jax.experimental import pallas as pl
from jax.experimental.pallas import tpu as pltpu
from jax.experimental.pallas import tpu_sc as plsc

N_CORES = 2
N_SUBCORES = 16
NW = N_CORES * N_SUBCORES
LANES = 16
BLK = 128
H = 4
C = 32
HC = H * C
NEG_SLOPE = 0.2
EPS = 1e-16
DEN_W = 16


def _project_body(x_ref, wl_ref, bl_ref, wr_ref, br_ref, xl_ref, xr_ref):
    xb = x_ref[...]
    xl_ref[...] = (
        jnp.dot(xb, wl_ref[...], preferred_element_type=jnp.float32) + bl_ref[...]
    )
    xr_ref[...] = (
        jnp.dot(xb, wr_ref[...], preferred_element_type=jnp.float32) + br_ref[...]
    )


def _project(x, W_l, b_l, W_r, b_r, bn):
    n, d = x.shape
    return pl.pallas_call(
        _project_body,
        grid=(n // bn,),
        in_specs=[
            pl.BlockSpec((bn, d), lambda i: (i, 0)),
            pl.BlockSpec((d, HC), lambda i: (0, 0)),
            pl.BlockSpec((HC,), lambda i: (0,)),
            pl.BlockSpec((d, HC), lambda i: (0, 0)),
            pl.BlockSpec((HC,), lambda i: (0,)),
        ],
        out_specs=[
            pl.BlockSpec((bn, HC), lambda i: (i, 0)),
            pl.BlockSpec((bn, HC), lambda i: (i, 0)),
        ],
        out_shape=[jax.ShapeDtypeStruct((n, HC), jnp.float32)] * 2,
    )(x, W_l, b_l, W_r, b_r)


def _edge_pass(xl, xr, src3, dst3, att_flat, z_den, z_out, n, nblk, etot):
    chunk = nblk * BLK
    den_tiles = n // 1000

    mesh = plsc.VectorSubcoreMesh(core_axis_name="c", subcore_axis_name="s")

    @functools.partial(
        pl.kernel,
        out_type=[
            jax.ShapeDtypeStruct((N_CORES, n, HC), jnp.float32),
            jax.ShapeDtypeStruct((N_CORES, n, DEN_W), jnp.float32),
        ],
        mesh=mesh,
        compiler_params=pltpu.CompilerParams(
            needs_layout_passes=False, use_tc_tiling_on_sc=False),
        scratch_types=[
            pltpu.VMEM((BLK,), jnp.int32),
            pltpu.VMEM((BLK,), jnp.int32),
            pltpu.VMEM((BLK, HC), jnp.float32),
            pltpu.VMEM((BLK, DEN_W), jnp.float32),
            pltpu.VMEM((HC,), jnp.float32),
            pltpu.VMEM_SHARED((n, HC), jnp.float32),
            pltpu.VMEM_SHARED((n, DEN_W), jnp.float32),
            pltpu.SemaphoreType.DMA,
            pltpu.SemaphoreType.DMA,
        ],
    )
    def body(xl_hbm, xr_hbm, src_hbm, dst_hbm, att_hbm, zden_hbm, zout_hbm,
             part_hbm, den_hbm,
             src_v, dst_v, xm_s, ea_s, att_v, out_sh, den_sh,
             sem1, sem2):
        cid = lax.axis_index("c")
        sid = lax.axis_index("s")
        wid = sid * N_CORES + cid

        pltpu.sync_copy(zout_hbm, out_sh.at[pl.ds(sid * 624, 640)])
        pltpu.sync_copy(zden_hbm, den_sh.at[pl.ds(sid * 624, 640)])

        pltpu.sync_copy(att_hbm, att_v)

        def zero_ea(i, carry):
            ea_s[i, :] = jnp.zeros((DEN_W,), jnp.float32)
            return carry
        lax.fori_loop(0, BLK, zero_ea, 0)
        plsc.subcore_barrier()

        base = wid * chunk
        iota16 = lax.iota(jnp.int32, LANES)

        def blk_body(b, carry):
            ci1 = pltpu.async_copy(src_hbm.at[wid, b], src_v, sem1)
            ci2 = pltpu.async_copy(dst_hbm.at[wid, b], dst_v, sem2)
            ci1.wait()
            ci2.wait()
            pltpu.sync_copy(xl_hbm.at[src_v], xm_s)
            pltpu.sync_copy(xr_hbm.at[dst_v], xm_s, add=True)

            def sub_body(t, carry2):
                rows = t * LANES + iota16
                gids = base + b * BLK + rows
                valid = gids < etot
                for h in range(H):
                    av0 = att_v[pl.ds(h * C, LANES)]
                    av1 = att_v[pl.ds(h * C + LANES, LANES)]
                    acc = jnp.zeros((LANES,), jnp.float32)
                    for cc in range(C):
                        f = h * C + cc
                        att_s = av0[cc] if cc < LANES else av1[cc - LANES]
                        colv = jnp.full((LANES,), f, jnp.int32)
                        z = plsc.load_gather(xm_s, [rows, colv])
                        z = jnp.maximum(z, NEG_SLOPE * z)
                        acc = acc + z * att_s
                    ea = jnp.exp(acc)
                    ea = jnp.where(valid, ea, 0.0)
                    plsc.store_scatter(
                        ea_s, [rows, jnp.full((LANES,), h, jnp.int32)], ea)
                    for cc in range(C):
                        f = h * C + cc
                        colv = jnp.full((LANES,), f, jnp.int32)
                        v = plsc.load_gather(xm_s, [rows, colv]) * ea
                        plsc.store_scatter(xm_s, [rows, colv], v)
                return carry2

            lax.fori_loop(0, BLK // LANES, sub_body, 0)
            cs1 = pltpu.async_copy(xm_s, out_sh.at[dst_v], sem1, add=True)
            cs2 = pltpu.async_copy(ea_s, den_sh.at[dst_v], sem2, add=True)
            cs1.wait()
            cs2.wait()
            return carry

        lax.fori_loop(0, nblk, blk_body, 0)
        plsc.subcore_barrier()

        pltpu.sync_copy(
            out_sh.at[pl.ds(sid * 624, 640)],
            part_hbm.at[cid, pl.ds(sid * 624, 640), :])
        pltpu.sync_copy(
            den_sh.at[pl.ds(sid * 624, 640)],
            den_hbm.at[cid, pl.ds(sid * 624, 640), :])

    return body(xl, xr, src3, dst3, att_flat, z_den, z_out)


def _finalize_body(p0_ref, p1_ref, d0_ref, d1_ref, xr_ref, bias_ref, o_ref):
    den = d0_ref[...] + d1_ref[...]
    bn = den.shape[0]
    den_exp = jnp.concatenate(
        [jnp.broadcast_to(den[:, h][:, None], (bn, C)) for h in range(H)],
        axis=1)
    num = p0_ref[...] + p1_ref[...] - xr_ref[...] * den_exp
    o_ref[...] = num / (den_exp + EPS) + bias_ref[...]


def _finalize(p0, p1, d0, d1, xr, bias, bn):
    n = p0.shape[0]
    return pl.pallas_call(
        _finalize_body,
        grid=(n // bn,),
        in_specs=[
            pl.BlockSpec((bn, HC), lambda i: (i, 0)),
            pl.BlockSpec((bn, HC), lambda i: (i, 0)),
            pl.BlockSpec((bn, DEN_W), lambda i: (i, 0)),
            pl.BlockSpec((bn, DEN_W), lambda i: (i, 0)),
            pl.BlockSpec((bn, HC), lambda i: (i, 0)),
            pl.BlockSpec((HC,), lambda i: (0,)),
        ],
        out_specs=pl.BlockSpec((bn, HC), lambda i: (i, 0)),
        out_shape=jax.ShapeDtypeStruct((n, HC), jnp.float32),
    )(p0, p1, d0, d1, xr, bias)


def kernel(x, edge_index, W_l, b_l, W_r, b_r, att, bias):
    n, d = x.shape
    e = edge_index.shape[1]
    etot = e + n
    nblk = -(-etot // (NW * BLK))
    chunk = nblk * BLK
    epad = NW * chunk

    idt = edge_index.dtype
    loop_idx = jnp.arange(n, dtype=idt)
    pad = jnp.zeros((epad - etot,), idt)
    src3 = jnp.concatenate([edge_index[0], loop_idx, pad]).reshape(NW, nblk, BLK)
    dst3 = jnp.concatenate([edge_index[1], loop_idx, pad]).reshape(NW, nblk, BLK)

    xl, xr = _project(x, W_l, b_l, W_r, b_r, bn=1000)

    z_den = jnp.zeros((640, DEN_W), jnp.float32)
    z_out = jnp.zeros((640, HC), jnp.float32)
    part, den = _edge_pass(xl, xr, src3, dst3, att.reshape(HC), z_den, z_out,
                           n, nblk, etot)

    return _finalize(part[0], part[1], den[0], den[1], xr, bias, bn=1000)


# speedup vs baseline: 35.5346x; 2.2848x over previous
import functools

import jax
import jax.numpy as jnp
from jax import lax
from jax.experimental import pallas as pl
from jax.experimental.pallas import tpu as pltpu
from jax.experimental.pallas import tpu_sc as plsc

N_CORES = 2
N_SUBCORES = 16
NW = N_CORES * N_SUBCORES
LANES = 16
BLK = 128
H = 4
C = 32
HC = H * C
NEG_SLOPE = 0.2
EPS = 1e-16
DEN_W = 16


def _project_body(x_ref, wl_ref, bl_ref, wr_ref, br_ref, xl_ref, xr_ref):
    xb = x_ref[...]
    xl_ref[...] = (
        jnp.dot(xb, wl_ref[...], preferred_element_type=jnp.float32) + bl_ref[...]
    )
    xr_ref[...] = (
        jnp.dot(xb, wr_ref[...], preferred_element_type=jnp.float32) + br_ref[...]
    )


def _project(x, W_l, b_l, W_r, b_r, bn):
    n, d = x.shape
    return pl.pallas_call(
        _project_body,
        grid=(n // bn,),
        in_specs=[
            pl.BlockSpec((bn, d), lambda i: (i, 0)),
            pl.BlockSpec((d, HC), lambda i: (0, 0)),
            pl.BlockSpec((HC,), lambda i: (0,)),
            pl.BlockSpec((d, HC), lambda i: (0, 0)),
            pl.BlockSpec((HC,), lambda i: (0,)),
        ],
        out_specs=[
            pl.BlockSpec((bn, HC), lambda i: (i, 0)),
            pl.BlockSpec((bn, HC), lambda i: (i, 0)),
        ],
        out_shape=[jax.ShapeDtypeStruct((n, HC), jnp.float32)] * 2,
    )(x, W_l, b_l, W_r, b_r)


def _edge_pass(xl, xr, src3, dst3, att_flat, z_den, z_out, n, nblk, etot):
    chunk = nblk * BLK
    den_tiles = n // 1000

    mesh = plsc.VectorSubcoreMesh(core_axis_name="c", subcore_axis_name="s")

    @functools.partial(
        pl.kernel,
        out_type=[
            jax.ShapeDtypeStruct((N_CORES, n, HC), jnp.float32),
            jax.ShapeDtypeStruct((N_CORES, n, DEN_W), jnp.float32),
        ],
        mesh=mesh,
        compiler_params=pltpu.CompilerParams(
            needs_layout_passes=False, use_tc_tiling_on_sc=False),
        scratch_types=[
            pltpu.VMEM((BLK,), jnp.int32),
            pltpu.VMEM((BLK,), jnp.int32),
            pltpu.VMEM((BLK, HC), jnp.float32),
            pltpu.VMEM((BLK, DEN_W), jnp.float32),
            pltpu.VMEM((HC,), jnp.float32),
            pltpu.VMEM_SHARED((n, HC), jnp.float32),
            pltpu.VMEM_SHARED((n, DEN_W), jnp.float32),
            pltpu.SemaphoreType.DMA,
            pltpu.SemaphoreType.DMA,
        ],
    )
    def body(xl_hbm, xr_hbm, src_hbm, dst_hbm, att_hbm, zden_hbm, zout_hbm,
             part_hbm, den_hbm,
             src_v, dst_v, xm_s, ea_s, att_v, out_sh, den_sh,
             sem1, sem2):
        cid = lax.axis_index("c")
        sid = lax.axis_index("s")
        wid = sid * N_CORES + cid

        pltpu.sync_copy(zout_hbm, out_sh.at[pl.ds(sid * 624, 640)])
        pltpu.sync_copy(zden_hbm, den_sh.at[pl.ds(sid * 624, 640)])

        pltpu.sync_copy(att_hbm, att_v)

        def zero_ea(i, carry):
            ea_s[i, :] = jnp.zeros((DEN_W,), jnp.float32)
            return carry
        lax.fori_loop(0, BLK, zero_ea, 0)
        plsc.subcore_barrier()

        base = wid * chunk
        iota16 = lax.iota(jnp.int32, LANES)

        def blk_body(b, carry):
            ci1 = pltpu.async_copy(src_hbm.at[wid, b], src_v, sem1)
            ci2 = pltpu.async_copy(dst_hbm.at[wid, b], dst_v, sem2)
            ci1.wait()
            ci2.wait()
            pltpu.sync_copy(xl_hbm.at[src_v], xm_s)
            pltpu.sync_copy(xr_hbm.at[dst_v], xm_s, add=True)

            def sub_body(t, carry2):
                rows = t * LANES + iota16
                gids = base + b * BLK + rows
                valid = gids < etot
                eas = []
                for h in range(H):
                    av0 = att_v[pl.ds(h * C, LANES)]
                    av1 = att_v[pl.ds(h * C + LANES, LANES)]
                    acc = jnp.zeros((LANES,), jnp.float32)
                    for cc in range(C):
                        f = h * C + cc
                        att_s = av0[cc] if cc < LANES else av1[cc - LANES]
                        colv = jnp.full((LANES,), f, jnp.int32)
                        z = plsc.load_gather(xm_s, [rows, colv])
                        z = jnp.maximum(z, NEG_SLOPE * z)
                        acc = acc + z * att_s
                    ea = jnp.exp(acc)
                    ea = jnp.where(valid, ea, 0.0)
                    plsc.store_scatter(
                        ea_s, [rows, jnp.full((LANES,), h, jnp.int32)], ea)
                    eas.append(ea)
                row0 = t * LANES
                for e in range(LANES):
                    for h in range(H):
                        s = eas[h][e]
                        for j in range(2):
                            sl = pl.ds(h * C + j * LANES, LANES)
                            xm_s[row0 + e, sl] = xm_s[row0 + e, sl] * s
                return carry2

            lax.fori_loop(0, BLK // LANES, sub_body, 0)
            cs1 = pltpu.async_copy(xm_s, out_sh.at[dst_v], sem1, add=True)
            cs2 = pltpu.async_copy(ea_s, den_sh.at[dst_v], sem2, add=True)
            cs1.wait()
            cs2.wait()
            return carry

        lax.fori_loop(0, nblk, blk_body, 0)
        plsc.subcore_barrier()

        pltpu.sync_copy(
            out_sh.at[pl.ds(sid * 624, 640)],
            part_hbm.at[cid, pl.ds(sid * 624, 640), :])
        pltpu.sync_copy(
            den_sh.at[pl.ds(sid * 624, 640)],
            den_hbm.at[cid, pl.ds(sid * 624, 640), :])

    return body(xl, xr, src3, dst3, att_flat, z_den, z_out)


def _finalize_body(p0_ref, p1_ref, d0_ref, d1_ref, xr_ref, bias_ref, o_ref):
    den = d0_ref[...] + d1_ref[...]
    bn = den.shape[0]
    den_exp = jnp.concatenate(
        [jnp.broadcast_to(den[:, h][:, None], (bn, C)) for h in range(H)],
        axis=1)
    num = p0_ref[...] + p1_ref[...] - xr_ref[...] * den_exp
    o_ref[...] = num / (den_exp + EPS) + bias_ref[...]


def _finalize(p0, p1, d0, d1, xr, bias, bn):
    n = p0.shape[0]
    return pl.pallas_call(
        _finalize_body,
        grid=(n // bn,),
        in_specs=[
            pl.BlockSpec((bn, HC), lambda i: (i, 0)),
            pl.BlockSpec((bn, HC), lambda i: (i, 0)),
            pl.BlockSpec((bn, DEN_W), lambda i: (i, 0)),
            pl.BlockSpec((bn, DEN_W), lambda i: (i, 0)),
            pl.BlockSpec((bn, HC), lambda i: (i, 0)),
            pl.BlockSpec((HC,), lambda i: (0,)),
        ],
        out_specs=pl.BlockSpec((bn, HC), lambda i: (i, 0)),
        out_shape=jax.ShapeDtypeStruct((n, HC), jnp.float32),
    )(p0, p1, d0, d1, xr, bias)


def kernel(x, edge_index, W_l, b_l, W_r, b_r, att, bias):
    n, d = x.shape
    e = edge_index.shape[1]
    etot = e + n
    nblk = -(-etot // (NW * BLK))
    chunk = nblk * BLK
    epad = NW * chunk

    idt = edge_index.dtype
    loop_idx = jnp.arange(n, dtype=idt)
    pad = jnp.zeros((epad - etot,), idt)
    src3 = jnp.concatenate([edge_index[0], loop_idx, pad]).reshape(NW, nblk, BLK)
    dst3 = jnp.concatenate([edge_index[1], loop_idx, pad]).reshape(NW, nblk, BLK)

    xl, xr = _project(x, W_l, b_l, W_r, b_r, bn=1000)

    z_den = jnp.zeros((640, DEN_W), jnp.float32)
    z_out = jnp.zeros((640, HC), jnp.float32)
    part, den = _edge_pass(xl, xr, src3, dst3, att.reshape(HC), z_den, z_out,
                           n, nblk, etot)

    return _finalize(part[0], part[1], den[0], den[1], xr, bias, bn=1000)
